# baseline (device time: 35655 ns/iter reference)
import jax
import jax.numpy as jnp
from jax import lax
from jax.experimental import pallas as pl
from jax.experimental.pallas import tpu as pltpu

N_DEV = 4


def kernel(x, Wq, K_ext, V_ext, Wo):
    B, Sq, D = x.shape
    _, Skv, Hl, Dh = K_ext.shape
    Dout = Wo.shape[1]
    Hd = Hl * Dh
    scale = 1.0 / (Dh ** 0.5)

    def body(x_ref, wq_ref, k_ref, v_ref, wo_ref, out_ref,
             comm_ref, send_sems, recv_sems):
        my_i = lax.axis_index("i")
        left = lax.rem(my_i + N_DEV - 1, N_DEV)
        right = lax.rem(my_i + 1, N_DEV)

        barrier_sem = pltpu.get_barrier_semaphore()
        for nbr in (left, right):
            pl.semaphore_signal(
                barrier_sem, inc=1,
                device_id=(nbr,), device_id_type=pl.DeviceIdType.MESH,
            )
        pl.semaphore_wait(barrier_sem, 2)

        wq_s = wq_ref[:, pl.ds(my_i * Hd, Hd)]
        wo_s = wo_ref[pl.ds(my_i * Hd, Hd), :]
        for b in range(B):
            xb = x_ref[b]
            qb = jnp.dot(xb, wq_s, preferred_element_type=jnp.float32)
            kb = k_ref[b]
            vb = v_ref[b]
            ctx_cols = []
            for h in range(Hl):
                qh = qb[:, h * Dh:(h + 1) * Dh]
                kh = kb[:, h, :]
                vh = vb[:, h, :]
                s = lax.dot_general(
                    qh, kh, (((1,), (1,)), ((), ())),
                    preferred_element_type=jnp.float32,
                ) * scale
                s = s - jnp.max(s, axis=1, keepdims=True)
                w = jnp.exp(s)
                w = w / jnp.sum(w, axis=1, keepdims=True)
                ctx_cols.append(
                    jnp.dot(w, vh, preferred_element_type=jnp.float32))
            ctx = jnp.concatenate(ctx_cols, axis=1)
            partial = jnp.dot(ctx, wo_s, preferred_element_type=jnp.float32)
            out_ref[b] = partial
            comm_ref[0, b] = partial

        for hop in range(N_DEV - 1):
            rdma = pltpu.make_async_remote_copy(
                src_ref=comm_ref.at[hop],
                dst_ref=comm_ref.at[hop + 1],
                send_sem=send_sems.at[hop],
                recv_sem=recv_sems.at[hop],
                device_id=(right,),
                device_id_type=pl.DeviceIdType.MESH,
            )
            rdma.start()
            rdma.wait()
            out_ref[...] += comm_ref[hop + 1]

    return pl.pallas_call(
        body,
        out_shape=jax.ShapeDtypeStruct((B, Sq, Dout), jnp.float32),
        in_specs=[pl.BlockSpec(memory_space=pltpu.VMEM)] * 5,
        out_specs=pl.BlockSpec(memory_space=pltpu.VMEM),
        scratch_shapes=[
            pltpu.VMEM((N_DEV, B, Sq, Dout), jnp.float32),
            pltpu.SemaphoreType.DMA((N_DEV - 1,)),
            pltpu.SemaphoreType.DMA((N_DEV - 1,)),
        ],
        compiler_params=pltpu.CompilerParams(collective_id=0),
    )(x, Wq, K_ext, V_ext, Wo)


# device time: 22378 ns/iter; 1.5933x vs baseline; 1.5933x over previous
import jax
import jax.numpy as jnp
from jax import lax
from jax.experimental import pallas as pl
from jax.experimental.pallas import tpu as pltpu

N_DEV = 4


def kernel(x, Wq, K_ext, V_ext, Wo):
    B, Sq, D = x.shape
    _, Skv, Hl, Dh = K_ext.shape
    Dout = Wo.shape[1]
    Hd = Hl * Dh
    Dhalf = Dout // 2
    scale = 1.0 / (Dh ** 0.5)

    def body(x_ref, wq_ref, k_ref, v_ref, wo_ref, out_ref,
             send_ref, acc_ref, recv_ref, send_sems, recv_sems):
        my_i = lax.axis_index("i")
        part_a = my_i ^ 1
        part_b = my_i ^ 3

        barrier_sem = pltpu.get_barrier_semaphore()
        for nbr in (part_a, part_b):
            pl.semaphore_signal(
                barrier_sem, inc=1,
                device_id=(nbr,), device_id_type=pl.DeviceIdType.MESH,
            )
        pl.semaphore_wait(barrier_sem, 2)

        wq_s = wq_ref[:, pl.ds(my_i * Hd, Hd)]
        wo_s = wo_ref[pl.ds(my_i * Hd, Hd), :]
        for b in range(B):
            xb = x_ref[b]
            qb = jnp.dot(xb, wq_s, preferred_element_type=jnp.float32)
            kb = k_ref[b]
            vb = v_ref[b]
            ctx_cols = []
            for h in range(Hl):
                qh = qb[:, h * Dh:(h + 1) * Dh]
                kh = kb[:, h, :]
                vh = vb[:, h, :]
                s = lax.dot_general(
                    qh, kh, (((1,), (1,)), ((), ())),
                    preferred_element_type=jnp.float32,
                ) * scale
                s = s - jnp.max(s, axis=1, keepdims=True)
                w = jnp.exp(s)
                w = w / jnp.sum(w, axis=1, keepdims=True)
                ctx_cols.append(
                    jnp.dot(w, vh, preferred_element_type=jnp.float32))
            ctx = jnp.concatenate(ctx_cols, axis=1)
            partial = jnp.dot(ctx, wo_s, preferred_element_type=jnp.float32)
            send_ref[0, b] = partial[:, :Dhalf]
            send_ref[1, b] = partial[:, Dhalf:]

        st1 = []
        for slot, tgt in ((0, part_a), (1, part_b)):
            rdma = pltpu.make_async_remote_copy(
                src_ref=send_ref.at[slot],
                dst_ref=recv_ref.at[slot],
                send_sem=send_sems.at[slot],
                recv_sem=recv_sems.at[slot],
                device_id=(tgt,),
                device_id_type=pl.DeviceIdType.MESH,
            )
            rdma.start()
            st1.append(rdma)
        for rdma in st1:
            rdma.wait()
        acc_ref[0] = send_ref[0] + recv_ref[0]
        acc_ref[1] = send_ref[1] + recv_ref[1]

        st2 = []
        for slot, tgt in ((0, part_b), (1, part_a)):
            rdma = pltpu.make_async_remote_copy(
                src_ref=acc_ref.at[slot],
                dst_ref=recv_ref.at[slot + 2],
                send_sem=send_sems.at[slot + 2],
                recv_sem=recv_sems.at[slot + 2],
                device_id=(tgt,),
                device_id_type=pl.DeviceIdType.MESH,
            )
            rdma.start()
            st2.append(rdma)
        for rdma in st2:
            rdma.wait()
        out_ref[:, :, :Dhalf] = acc_ref[0] + recv_ref[2]
        out_ref[:, :, Dhalf:] = acc_ref[1] + recv_ref[3]

    return pl.pallas_call(
        body,
        out_shape=jax.ShapeDtypeStruct((B, Sq, Dout), jnp.float32),
        in_specs=[pl.BlockSpec(memory_space=pltpu.VMEM)] * 5,
        out_specs=pl.BlockSpec(memory_space=pltpu.VMEM),
        scratch_shapes=[
            pltpu.VMEM((2, B, Sq, Dhalf), jnp.float32),
            pltpu.VMEM((2, B, Sq, Dhalf), jnp.float32),
            pltpu.VMEM((4, B, Sq, Dhalf), jnp.float32),
            pltpu.SemaphoreType.DMA((4,)),
            pltpu.SemaphoreType.DMA((4,)),
        ],
        compiler_params=pltpu.CompilerParams(collective_id=0),
    )(x, Wq, K_ext, V_ext, Wo)


# device time: 21927 ns/iter; 1.6261x vs baseline; 1.0206x over previous
import jax
import jax.numpy as jnp
from jax import lax
from jax.experimental import pallas as pl
from jax.experimental.pallas import tpu as pltpu

N_DEV = 4


def kernel(x, Wq, K_ext, V_ext, Wo):
    B, Sq, D = x.shape
    _, Skv, Hl, Dh = K_ext.shape
    Dout = Wo.shape[1]
    Hd = Hl * Dh
    Dhalf = Dout // 2
    M = B * Sq
    scale = 1.0 / (Dh ** 0.5)

    def body(x_ref, wq_ref, k_ref, v_ref, wo_ref, out_ref,
             send_ref, acc_ref, recv_ref, send_sems, recv_sems):
        my_i = lax.axis_index("i")
        part_a = my_i ^ 1
        part_b = my_i ^ 3

        barrier_sem = pltpu.get_barrier_semaphore()
        for nbr in (part_a, part_b):
            pl.semaphore_signal(
                barrier_sem, inc=1,
                device_id=(nbr,), device_id_type=pl.DeviceIdType.MESH,
            )
        pl.semaphore_wait(barrier_sem, 2)

        wq_s = wq_ref[:, pl.ds(my_i * Hd, Hd)]
        q_all = jnp.dot(x_ref[...], wq_s, preferred_element_type=jnp.float32)

        ctx_rows = []
        for b in range(B):
            kb = k_ref[b]
            vb = v_ref[b]
            ctx_cols = []
            for h in range(Hl):
                qh = q_all[b * Sq:(b + 1) * Sq, h * Dh:(h + 1) * Dh]
                kh = kb[:, h * Dh:(h + 1) * Dh]
                vh = vb[:, h * Dh:(h + 1) * Dh]
                s = lax.dot_general(
                    qh, kh, (((1,), (1,)), ((), ())),
                    preferred_element_type=jnp.float32,
                ) * scale
                s = s - jnp.max(s, axis=1, keepdims=True)
                w = jnp.exp(s)
                w = w / jnp.sum(w, axis=1, keepdims=True)
                ctx_cols.append(
                    jnp.dot(w, vh, preferred_element_type=jnp.float32))
            ctx_rows.append(jnp.concatenate(ctx_cols, axis=1))
        ctx = jnp.concatenate(ctx_rows, axis=0)

        wo_s = wo_ref[pl.ds(my_i * Hd, Hd), :]

        def exchange(slot, src, tgt):
            rdma = pltpu.make_async_remote_copy(
                src_ref=src,
                dst_ref=recv_ref.at[slot],
                send_sem=send_sems.at[slot],
                recv_sem=recv_sems.at[slot],
                device_id=(tgt,),
                device_id_type=pl.DeviceIdType.MESH,
            )
            rdma.start()
            return rdma

        send_ref[0] = jnp.dot(ctx, wo_s[:, :Dhalf],
                              preferred_element_type=jnp.float32)
        a1 = exchange(0, send_ref.at[0], part_a)
        send_ref[1] = jnp.dot(ctx, wo_s[:, Dhalf:],
                              preferred_element_type=jnp.float32)
        b1 = exchange(1, send_ref.at[1], part_b)

        a1.wait()
        acc_ref[0] = send_ref[0] + recv_ref[0]
        a2 = exchange(2, acc_ref.at[0], part_b)
        b1.wait()
        acc_ref[1] = send_ref[1] + recv_ref[1]
        b2 = exchange(3, acc_ref.at[1], part_a)

        a2.wait()
        out_ref[:, :Dhalf] = acc_ref[0] + recv_ref[2]
        b2.wait()
        out_ref[:, Dhalf:] = acc_ref[1] + recv_ref[3]

    out2d = pl.pallas_call(
        body,
        out_shape=jax.ShapeDtypeStruct((M, Dout), jnp.float32),
        in_specs=[pl.BlockSpec(memory_space=pltpu.VMEM)] * 5,
        out_specs=pl.BlockSpec(memory_space=pltpu.VMEM),
        scratch_shapes=[
            pltpu.VMEM((2, M, Dhalf), jnp.float32),
            pltpu.VMEM((2, M, Dhalf), jnp.float32),
            pltpu.VMEM((4, M, Dhalf), jnp.float32),
            pltpu.SemaphoreType.DMA((4,)),
            pltpu.SemaphoreType.DMA((4,)),
        ],
        compiler_params=pltpu.CompilerParams(collective_id=0),
    )(
        x.reshape(M, D),
        Wq,
        K_ext.reshape(B, Skv, Hd),
        V_ext.reshape(B, Skv, Hd),
        Wo,
    )
    return out2d.reshape(B, Sq, Dout)


# device time: 19117 ns/iter; 1.8651x vs baseline; 1.1470x over previous
import jax
import jax.numpy as jnp
from jax import lax
from jax.experimental import pallas as pl
from jax.experimental.pallas import tpu as pltpu

N_DEV = 4


def kernel(x, Wq, K_ext, V_ext, Wo):
    B, Sq, D = x.shape
    _, Skv, Hl, Dh = K_ext.shape
    Dout = Wo.shape[1]
    Hd = Hl * Dh
    Dhalf = Dout // 2
    M = B * Sq
    scale = 1.0 / (Dh ** 0.5)

    def body(x_ref, wq_ref, k_ref, v_ref, wo_ref, out_ref,
             send_ref, acc_ref, recv_ref, send_sems, recv_sems):
        my_i = lax.axis_index("i")
        part_a = my_i ^ 1
        part_b = my_i ^ 3

        barrier_sem = pltpu.get_barrier_semaphore()
        for nbr in (part_a, part_b):
            pl.semaphore_signal(
                barrier_sem, inc=1,
                device_id=(nbr,), device_id_type=pl.DeviceIdType.MESH,
            )
        pl.semaphore_wait(barrier_sem, 2)

        wq_s = wq_ref[:, pl.ds(my_i * Hd, Hd)]
        q_all = jnp.dot(x_ref[...], wq_s, preferred_element_type=jnp.float32)

        ctx_rows = []
        for b in range(B):
            kb = k_ref[b]
            vb = v_ref[b]
            ctx_cols = []
            for h in range(Hl):
                qh = q_all[b * Sq:(b + 1) * Sq, h * Dh:(h + 1) * Dh]
                kh = kb[:, h * Dh:(h + 1) * Dh]
                vh = vb[:, h * Dh:(h + 1) * Dh]
                s = lax.dot_general(
                    qh, kh, (((1,), (1,)), ((), ())),
                    preferred_element_type=jnp.float32,
                ) * scale
                s = s - jnp.max(s, axis=1, keepdims=True)
                w = jnp.exp(s)
                w = w / jnp.sum(w, axis=1, keepdims=True)
                ctx_cols.append(
                    jnp.dot(w, vh, preferred_element_type=jnp.float32))
            ctx_rows.append(jnp.concatenate(ctx_cols, axis=1))
        ctx = jnp.concatenate(ctx_rows, axis=0)

        wo_s = wo_ref[pl.ds(my_i * Hd, Hd), :]

        def exchange(slot, src, tgt):
            rdma = pltpu.make_async_remote_copy(
                src_ref=src,
                dst_ref=recv_ref.at[slot],
                send_sem=send_sems.at[slot],
                recv_sem=recv_sems.at[slot],
                device_id=(tgt,),
                device_id_type=pl.DeviceIdType.MESH,
            )
            rdma.start()
            return rdma

        half_a = jnp.dot(ctx, wo_s[:, :Dhalf],
                         preferred_element_type=jnp.float32)
        send_ref[0] = half_a.astype(jnp.bfloat16)
        a1 = exchange(0, send_ref.at[0], part_a)
        half_b = jnp.dot(ctx, wo_s[:, Dhalf:],
                         preferred_element_type=jnp.float32)
        send_ref[1] = half_b.astype(jnp.bfloat16)
        b1 = exchange(1, send_ref.at[1], part_b)

        a1.wait()
        acc_a = half_a + recv_ref[0].astype(jnp.float32)
        acc_ref[0] = acc_a.astype(jnp.bfloat16)
        a2 = exchange(2, acc_ref.at[0], part_b)
        b1.wait()
        acc_b = half_b + recv_ref[1].astype(jnp.float32)
        acc_ref[1] = acc_b.astype(jnp.bfloat16)
        b2 = exchange(3, acc_ref.at[1], part_a)

        a2.wait()
        out_a = acc_a + recv_ref[2].astype(jnp.float32)
        out_ref[:, :, :Dhalf] = out_a.reshape(B, Sq, Dhalf)
        b2.wait()
        out_b = acc_b + recv_ref[3].astype(jnp.float32)
        out_ref[:, :, Dhalf:] = out_b.reshape(B, Sq, Dhalf)

    return pl.pallas_call(
        body,
        out_shape=jax.ShapeDtypeStruct((B, Sq, Dout), jnp.float32),
        in_specs=[pl.BlockSpec(memory_space=pltpu.VMEM)] * 5,
        out_specs=pl.BlockSpec(memory_space=pltpu.VMEM),
        scratch_shapes=[
            pltpu.VMEM((2, M, Dhalf), jnp.bfloat16),
            pltpu.VMEM((2, M, Dhalf), jnp.bfloat16),
            pltpu.VMEM((4, M, Dhalf), jnp.bfloat16),
            pltpu.SemaphoreType.DMA((4,)),
            pltpu.SemaphoreType.DMA((4,)),
        ],
        compiler_params=pltpu.CompilerParams(collective_id=0),
    )(
        x.reshape(M, D),
        Wq,
        K_ext.reshape(B, Skv, Hd),
        V_ext.reshape(B, Skv, Hd),
        Wo,
    )


# device time: 17516 ns/iter; 2.0356x vs baseline; 1.0914x over previous
import jax
import jax.numpy as jnp
from jax import lax
from jax.experimental import pallas as pl
from jax.experimental.pallas import tpu as pltpu

N_DEV = 4


def kernel(x, Wq, K_ext, V_ext, Wo):
    B, Sq, D = x.shape
    _, Skv, Hl, Dh = K_ext.shape
    Dout = Wo.shape[1]
    Hd = Hl * Dh
    Dhalf = Dout // 2
    M = B * Sq
    scale = 1.0 / (Dh ** 0.5)
    bf16 = jnp.bfloat16

    my_i_outer = lax.axis_index("i")

    def body(x_ref, wq_ref, k_ref, v_ref, wo_ref, out_ref,
             send_ref, acc_ref, recv_ref, send_sems, recv_sems):
        my_i = lax.axis_index("i")
        part_a = my_i ^ 1
        part_b = my_i ^ 3

        barrier_sem = pltpu.get_barrier_semaphore()
        for nbr in (part_a, part_b):
            pl.semaphore_signal(
                barrier_sem, inc=1,
                device_id=(nbr,), device_id_type=pl.DeviceIdType.MESH,
            )
        pl.semaphore_wait(barrier_sem, 2)

        q_all = jnp.dot(x_ref[...], wq_ref[...],
                        preferred_element_type=jnp.float32)
        q_all = q_all.astype(bf16)

        ctx_rows = []
        for b in range(B):
            kb = k_ref[b]
            vb = v_ref[b]
            ctx_cols = []
            for h in range(Hl):
                qh = q_all[b * Sq:(b + 1) * Sq, h * Dh:(h + 1) * Dh]
                kh = kb[:, h * Dh:(h + 1) * Dh]
                vh = vb[:, h * Dh:(h + 1) * Dh]
                s = lax.dot_general(
                    qh, kh, (((1,), (1,)), ((), ())),
                    preferred_element_type=jnp.float32,
                ) * scale
                s = s - jnp.max(s, axis=1, keepdims=True)
                w = jnp.exp(s)
                w = (w / jnp.sum(w, axis=1, keepdims=True)).astype(bf16)
                ctx_cols.append(
                    jnp.dot(w, vh, preferred_element_type=jnp.float32))
            ctx_rows.append(jnp.concatenate(ctx_cols, axis=1))
        ctx = jnp.concatenate(ctx_rows, axis=0).astype(bf16)

        def exchange(slot, src, tgt):
            rdma = pltpu.make_async_remote_copy(
                src_ref=src,
                dst_ref=recv_ref.at[slot],
                send_sem=send_sems.at[slot],
                recv_sem=recv_sems.at[slot],
                device_id=(tgt,),
                device_id_type=pl.DeviceIdType.MESH,
            )
            rdma.start()
            return rdma

        half_a = jnp.dot(ctx, wo_ref[:, :Dhalf],
                         preferred_element_type=jnp.float32)
        send_ref[0] = half_a.astype(bf16)
        a1 = exchange(0, send_ref.at[0], part_a)
        half_b = jnp.dot(ctx, wo_ref[:, Dhalf:],
                         preferred_element_type=jnp.float32)
        send_ref[1] = half_b.astype(bf16)
        b1 = exchange(1, send_ref.at[1], part_b)

        a1.wait()
        acc_a = half_a + recv_ref[0].astype(jnp.float32)
        acc_ref[0] = acc_a.astype(bf16)
        a2 = exchange(2, acc_ref.at[0], part_b)
        b1.wait()
        acc_b = half_b + recv_ref[1].astype(jnp.float32)
        acc_ref[1] = acc_b.astype(bf16)
        b2 = exchange(3, acc_ref.at[1], part_a)

        a2.wait()
        out_a = acc_a + recv_ref[2].astype(jnp.float32)
        out_ref[:, :, :Dhalf] = out_a.reshape(B, Sq, Dhalf)
        b2.wait()
        out_b = acc_b + recv_ref[3].astype(jnp.float32)
        out_ref[:, :, Dhalf:] = out_b.reshape(B, Sq, Dhalf)

    return pl.pallas_call(
        body,
        out_shape=jax.ShapeDtypeStruct((B, Sq, Dout), jnp.float32),
        in_specs=[pl.BlockSpec(memory_space=pltpu.VMEM)] * 5,
        out_specs=pl.BlockSpec(memory_space=pltpu.VMEM),
        scratch_shapes=[
            pltpu.VMEM((2, M, Dhalf), bf16),
            pltpu.VMEM((2, M, Dhalf), bf16),
            pltpu.VMEM((4, M, Dhalf), bf16),
            pltpu.SemaphoreType.DMA((4,)),
            pltpu.SemaphoreType.DMA((4,)),
        ],
        compiler_params=pltpu.CompilerParams(collective_id=0),
    )(
        x.reshape(M, D).astype(bf16),
        lax.dynamic_slice_in_dim(Wq, my_i_outer * Hd, Hd, 1).astype(bf16),
        K_ext.reshape(B, Skv, Hd).astype(bf16),
        V_ext.reshape(B, Skv, Hd).astype(bf16),
        lax.dynamic_slice_in_dim(Wo, my_i_outer * Hd, Hd, 0).astype(bf16),
    )
